# baseline (speedup 1.0000x reference)
import jax
import jax.numpy as jnp
from jax.experimental import pallas as pl
from jax.experimental.pallas import tpu as pltpu

_N = 1_000_000
_NCHUNK = 4
_CHUNK = _N // _NCHUNK


def _copy_body(in_hbm, out_hbm, *rest):
    bufs = rest[:_NCHUNK]
    in_sem, out_sem = rest[_NCHUNK], rest[_NCHUNK + 1]
    for i in range(_NCHUNK):
        pltpu.make_async_copy(in_hbm.at[i], bufs[i], in_sem.at[i]).start()
    for i in range(_NCHUNK):
        pltpu.make_async_copy(in_hbm.at[i], bufs[i], in_sem.at[i]).wait()
        pltpu.make_async_copy(bufs[i], out_hbm.at[i], out_sem.at[i]).start()
    for i in range(_NCHUNK):
        pltpu.make_async_copy(bufs[i], out_hbm.at[i], out_sem.at[i]).wait()


def kernel(goal_logits):
    x = goal_logits.reshape(_NCHUNK, _CHUNK)
    out = pl.pallas_call(
        _copy_body,
        out_shape=jax.ShapeDtypeStruct((_NCHUNK, _CHUNK), jnp.float32),
        in_specs=[pl.BlockSpec(memory_space=pl.ANY)],
        out_specs=pl.BlockSpec(memory_space=pl.ANY),
        scratch_shapes=(
            [pltpu.VMEM((_CHUNK,), jnp.float32) for _ in range(_NCHUNK)]
            + [pltpu.SemaphoreType.DMA((_NCHUNK,)),
               pltpu.SemaphoreType.DMA((_NCHUNK,))]
        ),
    )(x)
    return out.reshape(_N)


# speedup vs baseline: 4.0650x; 4.0650x over previous
import jax
import jax.numpy as jnp
from jax.experimental import pallas as pl
from jax.experimental.pallas import tpu as pltpu

_N = 1_000_000


def _copy_body(in_hbm, out_hbm, buf, in_sem, out_sem):
    pltpu.make_async_copy(in_hbm, buf, in_sem).start()
    pltpu.make_async_copy(in_hbm, buf, in_sem).wait()
    pltpu.make_async_copy(buf, out_hbm, out_sem).start()
    pltpu.make_async_copy(buf, out_hbm, out_sem).wait()


def kernel(goal_logits):
    return pl.pallas_call(
        _copy_body,
        out_shape=jax.ShapeDtypeStruct((_N,), jnp.float32),
        in_specs=[pl.BlockSpec(memory_space=pl.ANY)],
        out_specs=pl.BlockSpec(memory_space=pl.ANY),
        scratch_shapes=[
            pltpu.VMEM((_N,), jnp.float32),
            pltpu.SemaphoreType.DMA,
            pltpu.SemaphoreType.DMA,
        ],
    )(goal_logits)
